# baseline (speedup 1.0000x reference)
import functools

import jax
import jax.numpy as jnp
from jax import lax
from jax.experimental import pallas as pl
from jax.experimental.pallas import tpu as pltpu
from jax.experimental.pallas import tpu_sc as plsc

B = 16384
D = 32
V = 1000000
NC = 2
NS = 16
NW = NC * NS
BPW = B // NW
L = 16
NG = BPW // L
CH = 128
NCH = D * BPW // CH
QP = BPW // CH
WAVES = 16
CPW = NCH // WAVES


@functools.partial(
    pl.kernel,
    out_type=jax.ShapeDtypeStruct((B,), jnp.float32),
    mesh=plsc.VectorSubcoreMesh(core_axis_name="c", subcore_axis_name="s"),
    compiler_params=pltpu.CompilerParams(use_tc_tiling_on_sc=False),
    scratch_types=[
        pltpu.VMEM((BPW,), jnp.int32),
        pltpu.VMEM((BPW,), jnp.int32),
        pltpu.VMEM((NCH, CH), jnp.int32),
        pltpu.VMEM((NCH, CH), jnp.int32),
        pltpu.VMEM((D * BPW,), jnp.float32),
        pltpu.VMEM((D * BPW,), jnp.float32),
        pltpu.VMEM((BPW,), jnp.float32),
        pltpu.SemaphoreType.DMA,
    ],
)
def _mf_sc_kernel(xu_hbm, xv_hbm, ut_hbm, it_hbm, out_hbm,
                  idxu, idxv, widxu, widxv, ustage, istage, outv, sem):
    wid = lax.axis_index("s") * NC + lax.axis_index("c")
    base = wid * BPW

    pltpu.sync_copy(xu_hbm.at[pl.ds(base, BPW)], idxu)
    pltpu.sync_copy(xv_hbm.at[pl.ds(base, BPW)], idxv)

    def build_body(t, carry):
        off = t * L
        q = off // CH
        kk = off - q * CH
        vu = idxu[pl.ds(off, L)]
        vv = idxv[pl.ds(off, L)]
        for d in range(D):
            widxu[d * QP + q, pl.ds(kk, L)] = vu + d * V
            widxv[d * QP + q, pl.ds(kk, L)] = vv + d * V
        return carry

    lax.fori_loop(0, BPW // L, build_body, 0)

    def fire_wave(w):
        for j in range(CPW):
            c = w * CPW + j
            pltpu.async_copy(
                ut_hbm.at[widxu.at[c]], ustage.at[pl.ds(c * CH, CH)], sem)
            pltpu.async_copy(
                it_hbm.at[widxv.at[c]], istage.at[pl.ds(c * CH, CH)], sem)

    def drain_wave():
        pltpu.make_async_copy(
            ut_hbm.at[pl.ds(0, CPW * CH)], ustage.at[pl.ds(0, CPW * CH)], sem).wait()
        pltpu.make_async_copy(
            it_hbm.at[pl.ds(0, CPW * CH)], istage.at[pl.ds(0, CPW * CH)], sem).wait()

    fire_wave(0)

    def wave_body(w, carry):
        fire_wave(w)
        drain_wave()
        return carry

    lax.fori_loop(1, WAVES, wave_body, 0, unroll=False)
    drain_wave()

    def compute_body(g, carry):
        o = g * L
        acc = ustage[pl.ds(o, L)] * istage[pl.ds(o, L)]
        for d in range(1, D):
            acc = acc + ustage[pl.ds(d * BPW + o, L)] * istage[pl.ds(d * BPW + o, L)]
        outv[pl.ds(o, L)] = acc
        return carry

    lax.fori_loop(0, NG, compute_body, 0)

    pltpu.sync_copy(outv, out_hbm.at[pl.ds(base, BPW)])


def kernel(x, user_table, item_table):
    x = x.astype(jnp.int32)
    return _mf_sc_kernel(
        x[:, 0], x[:, 1],
        user_table.T.reshape(-1), item_table.T.reshape(-1))


# speedup vs baseline: 1.0031x; 1.0031x over previous
import functools

import jax
import jax.numpy as jnp
from jax import lax
from jax.experimental import pallas as pl
from jax.experimental.pallas import tpu as pltpu
from jax.experimental.pallas import tpu_sc as plsc

B = 16384
D = 32
V = 1000000
NC = 2
NS = 16
NW = NC * NS
BPW = B // NW
L = 16
NG = BPW // L
CH = 128
QP = BPW // CH


@functools.partial(
    pl.kernel,
    out_type=jax.ShapeDtypeStruct((B,), jnp.float32),
    mesh=plsc.VectorSubcoreMesh(core_axis_name="c", subcore_axis_name="s"),
    compiler_params=pltpu.CompilerParams(use_tc_tiling_on_sc=False),
    scratch_types=[
        pltpu.VMEM((QP, CH), jnp.int32),
        pltpu.VMEM((QP, CH), jnp.int32),
        pltpu.VMEM((D * BPW,), jnp.float32),
        pltpu.VMEM((D * BPW,), jnp.float32),
        pltpu.VMEM((BPW,), jnp.float32),
        pltpu.SemaphoreType.DMA,
    ],
)
def _mf_sc_kernel(xu_hbm, xv_hbm, ut_hbm, it_hbm, out_hbm,
                  idxu, idxv, ustage, istage, outv, sem):
    wid = lax.axis_index("s") * NC + lax.axis_index("c")
    base = wid * BPW

    pltpu.sync_copy(xu_hbm.at[pl.ds(wid * QP, QP)], idxu)
    pltpu.sync_copy(xv_hbm.at[pl.ds(wid * QP, QP)], idxv)

    def fire_row(d):
        for q in range(QP):
            o = d * BPW + q * CH
            pltpu.async_copy(
                ut_hbm.at[d].at[idxu.at[q]], ustage.at[pl.ds(o, CH)], sem)
            pltpu.async_copy(
                it_hbm.at[d].at[idxv.at[q]], istage.at[pl.ds(o, CH)], sem)

    def drain_row():
        pltpu.make_async_copy(
            ut_hbm.at[0, pl.ds(0, BPW)], ustage.at[pl.ds(0, BPW)], sem).wait()
        pltpu.make_async_copy(
            it_hbm.at[0, pl.ds(0, BPW)], istage.at[pl.ds(0, BPW)], sem).wait()

    fire_row(0)

    def row_body(d, carry):
        fire_row(d)
        drain_row()
        return carry

    lax.fori_loop(1, D, row_body, 0)
    drain_row()

    def compute_body(g, carry):
        o = g * L
        acc = ustage[pl.ds(o, L)] * istage[pl.ds(o, L)]
        for d in range(1, D):
            acc = acc + ustage[pl.ds(d * BPW + o, L)] * istage[pl.ds(d * BPW + o, L)]
        outv[pl.ds(o, L)] = acc
        return carry

    lax.fori_loop(0, NG, compute_body, 0)

    pltpu.sync_copy(outv, out_hbm.at[pl.ds(base, BPW)])


def kernel(x, user_table, item_table):
    x = x.astype(jnp.int32)
    xu = x[:, 0].reshape(NW * QP, CH)
    xv = x[:, 1].reshape(NW * QP, CH)
    return _mf_sc_kernel(xu, xv, user_table.T, item_table.T)


# speedup vs baseline: 24.4300x; 24.3554x over previous
import functools

import jax
import jax.numpy as jnp
from jax import lax
from jax.experimental import pallas as pl
from jax.experimental.pallas import tpu as pltpu
from jax.experimental.pallas import tpu_sc as plsc

B = 16384
D = 32
V = 1000000
NC = 2
NS = 16
NW = NC * NS
BPW = B // NW
L = 16
NG = BPW // L
RING = 4


@functools.partial(
    pl.kernel,
    out_type=jax.ShapeDtypeStruct((B,), jnp.float32),
    mesh=plsc.VectorSubcoreMesh(core_axis_name="c", subcore_axis_name="s"),
    compiler_params=pltpu.CompilerParams(needs_layout_passes=False),
    scratch_types=[
        pltpu.VMEM((BPW,), jnp.int32),
        pltpu.VMEM((BPW,), jnp.int32),
        pltpu.VMEM((RING, D, 128), jnp.float32),
        pltpu.VMEM((RING, D, 128), jnp.float32),
        pltpu.VMEM((D * BPW,), jnp.float32),
        pltpu.VMEM((D * BPW,), jnp.float32),
        pltpu.VMEM((BPW,), jnp.float32),
        pltpu.SemaphoreType.DMA,
    ],
)
def _mf_sc_kernel(xu_hbm, xv_hbm, ut_hbm, it_hbm, out_hbm,
                  idxu, idxv, ublk, iblk, ustage, istage, outv, sem):
    wid = lax.axis_index("s") * NC + lax.axis_index("c")
    base = wid * BPW

    pltpu.sync_copy(xu_hbm.at[pl.ds(base, BPW)], idxu)
    pltpu.sync_copy(xv_hbm.at[pl.ds(base, BPW)], idxv)

    iota = jax.lax.iota(jnp.int32, L)

    def fire(i_u, i_v, slot):
        cu = pl.multiple_of((i_u // 128) * 128, 128)
        cv = pl.multiple_of((i_v // 128) * 128, 128)
        pltpu.async_copy(
            ut_hbm.at[:, pl.ds(cu, 128)], ublk.at[slot], sem)
        pltpu.async_copy(
            it_hbm.at[:, pl.ds(cv, 128)], iblk.at[slot], sem)

    def wait_two():
        pltpu.make_async_copy(
            ut_hbm.at[:, pl.ds(0, 128)], ublk.at[0], sem).wait()
        pltpu.make_async_copy(
            it_hbm.at[:, pl.ds(0, 128)], iblk.at[0], sem).wait()

    def extract(i_u, i_v, slot, r):
        lu = jnp.full((L,), i_u % 128, jnp.int32)
        lv = jnp.full((L,), i_v % 128, jnp.int32)
        sfull = jnp.full((L,), slot, jnp.int32)
        for h in range(2):
            rows = iota + h * L
            uu = plsc.load_gather(ublk, [sfull, rows, lu])
            vv = plsc.load_gather(iblk, [sfull, rows, lv])
            plsc.store_scatter(ustage, [rows * BPW + r], uu)
            plsc.store_scatter(istage, [rows * BPW + r], vv)

    v_u0 = idxu[pl.ds(0, L)]
    v_v0 = idxv[pl.ds(0, L)]
    for k in range(RING):
        fire(v_u0[k], v_v0[k], k)

    def group_body(g, carry):
        o = g * L
        vu = idxu[pl.ds(o, L)]
        vv = idxv[pl.ds(o, L)]
        on = jnp.minimum(o + L, BPW - L)
        vu_n = idxu[pl.ds(on, L)]
        vv_n = idxv[pl.ds(on, L)]
        last = g == NG - 1
        for k in range(L):
            r = o + k
            slot = k % RING
            wait_two()
            extract(vu[k], vv[k], slot, r)
            kn = (k + RING) % L
            if k < L - RING:
                fire(vu[k + RING], vv[k + RING], slot)
            else:
                nu = jnp.where(last, vu[k], vu_n[kn])
                nv = jnp.where(last, vv[k], vv_n[kn])

                @pl.when(jnp.logical_not(last))
                def _():
                    fire(nu, nv, slot)
        return carry

    lax.fori_loop(0, NG, group_body, 0)

    def compute_body(g, carry):
        o = g * L
        acc = ustage[pl.ds(o, L)] * istage[pl.ds(o, L)]
        for d in range(1, D):
            acc = acc + ustage[pl.ds(d * BPW + o, L)] * istage[pl.ds(d * BPW + o, L)]
        outv[pl.ds(o, L)] = acc
        return carry

    lax.fori_loop(0, NG, compute_body, 0)

    pltpu.sync_copy(outv, out_hbm.at[pl.ds(base, BPW)])


def kernel(x, user_table, item_table):
    x = x.astype(jnp.int32)
    return _mf_sc_kernel(x[:, 0], x[:, 1], user_table.T, item_table.T)


# speedup vs baseline: 25.0256x; 1.0244x over previous
import functools

import jax
import jax.numpy as jnp
from jax import lax
from jax.experimental import pallas as pl
from jax.experimental.pallas import tpu as pltpu
from jax.experimental.pallas import tpu_sc as plsc

B = 16384
D = 32
V = 1000000
NC = 2
NS = 16
NW = NC * NS
BPW = B // NW
L = 16
NG = BPW // L
RING = 8


@functools.partial(
    pl.kernel,
    out_type=jax.ShapeDtypeStruct((B,), jnp.float32),
    mesh=plsc.VectorSubcoreMesh(core_axis_name="c", subcore_axis_name="s"),
    compiler_params=pltpu.CompilerParams(needs_layout_passes=False),
    scratch_types=[
        pltpu.VMEM((BPW,), jnp.int32),
        pltpu.VMEM((BPW,), jnp.int32),
        pltpu.VMEM((RING, D, 128), jnp.float32),
        pltpu.VMEM((RING, D, 128), jnp.float32),
        pltpu.VMEM((D * BPW,), jnp.float32),
        pltpu.VMEM((D * BPW,), jnp.float32),
        pltpu.VMEM((BPW,), jnp.float32),
        pltpu.SemaphoreType.DMA,
        pltpu.SemaphoreType.DMA,
    ],
)
def _mf_sc_kernel(xu_hbm, xv_hbm, ut_hbm, it_hbm, out_hbm,
                  idxu, idxv, ublk, iblk, ustage, istage, outv, semu, semv):
    wid = lax.axis_index("s") * NC + lax.axis_index("c")
    base = wid * BPW

    pltpu.sync_copy(xu_hbm.at[pl.ds(base, BPW)], idxu)
    pltpu.sync_copy(xv_hbm.at[pl.ds(base, BPW)], idxv)

    iota = jax.lax.iota(jnp.int32, L)

    def fire(i_u, i_v, slot):
        cu = pl.multiple_of((i_u // 128) * 128, 128)
        cv = pl.multiple_of((i_v // 128) * 128, 128)
        pltpu.async_copy(
            ut_hbm.at[:, pl.ds(cu, 128)], ublk.at[slot], semu)
        pltpu.async_copy(
            it_hbm.at[:, pl.ds(cv, 128)], iblk.at[slot], semv)

    def wait_two():
        pltpu.make_async_copy(
            ut_hbm.at[:, pl.ds(0, 128)], ublk.at[0], semu).wait()
        pltpu.make_async_copy(
            it_hbm.at[:, pl.ds(0, 128)], iblk.at[0], semv).wait()

    def extract(i_u, i_v, slot, r):
        lu = jnp.full((L,), i_u % 128, jnp.int32)
        lv = jnp.full((L,), i_v % 128, jnp.int32)
        sfull = jnp.full((L,), slot, jnp.int32)
        for h in range(2):
            rows = iota + h * L
            uu = plsc.load_gather(ublk, [sfull, rows, lu])
            vv = plsc.load_gather(iblk, [sfull, rows, lv])
            plsc.store_scatter(ustage, [rows * BPW + r], uu)
            plsc.store_scatter(istage, [rows * BPW + r], vv)

    v_u0 = idxu[pl.ds(0, L)]
    v_v0 = idxv[pl.ds(0, L)]
    for k in range(RING):
        fire(v_u0[k], v_v0[k], k)

    def group_body(g, carry):
        o = g * L
        vu = idxu[pl.ds(o, L)]
        vv = idxv[pl.ds(o, L)]
        on = jnp.minimum(o + L, BPW - L)
        vu_n = idxu[pl.ds(on, L)]
        vv_n = idxv[pl.ds(on, L)]
        last = g == NG - 1
        for k in range(L):
            r = o + k
            slot = k % RING
            wait_two()
            extract(vu[k], vv[k], slot, r)
            kn = (k + RING) % L
            if k < L - RING:
                fire(vu[k + RING], vv[k + RING], slot)
            else:
                nu = jnp.where(last, vu[k], vu_n[kn])
                nv = jnp.where(last, vv[k], vv_n[kn])

                @pl.when(jnp.logical_not(last))
                def _():
                    fire(nu, nv, slot)
        return carry

    lax.fori_loop(0, NG, group_body, 0)

    def compute_body(g, carry):
        o = g * L
        acc = ustage[pl.ds(o, L)] * istage[pl.ds(o, L)]
        for d in range(1, D):
            acc = acc + ustage[pl.ds(d * BPW + o, L)] * istage[pl.ds(d * BPW + o, L)]
        outv[pl.ds(o, L)] = acc
        return carry

    lax.fori_loop(0, NG, compute_body, 0)

    pltpu.sync_copy(outv, out_hbm.at[pl.ds(base, BPW)])


def kernel(x, user_table, item_table):
    x = x.astype(jnp.int32)
    return _mf_sc_kernel(x[:, 0], x[:, 1], user_table.T, item_table.T)
